# Initial kernel scaffold; baseline (speedup 1.0000x reference)
import jax
import jax.numpy as jnp
from jax import lax
from jax.experimental import pallas as pl
from jax.experimental.pallas import tpu as pltpu
from jax.experimental.pallas import tpu_sc as plsc

N = 10000
E = 320000
D = 128

N_PAD = 10240
DUMMY_ROW = N
CHUNK = 128
N_WORKERS = 32
CHUNKS_PER_TILE = 79
EDGES_PER_TILE = CHUNKS_PER_TILE * CHUNK
E_PAD = N_WORKERS * EDGES_PER_TILE
ROWS_PER_TILE = N_PAD // 16


def _make_sc_agg(with_cnt: bool):
    mesh = plsc.VectorSubcoreMesh(core_axis_name="c", subcore_axis_name="s")
    out_type = [jax.ShapeDtypeStruct((2, N_PAD, D), jnp.float32)]
    scratch = [
        pltpu.VMEM_SHARED((N_PAD, D), jnp.float32),
        pltpu.VMEM((1, CHUNK), jnp.int32),
        pltpu.VMEM((1, CHUNK), jnp.int32),
        pltpu.VMEM((CHUNK, D), jnp.float32),
        pltpu.VMEM((16, D), jnp.float32),
        pltpu.SemaphoreType.DMA,
    ]
    if with_cnt:
        out_type.append(jax.ShapeDtypeStruct((2, N_PAD), jnp.float32))
        scratch += [
            pltpu.VMEM_SHARED((N_PAD,), jnp.float32),
            pltpu.VMEM((CHUNK,), jnp.float32),
        ]

    def body(h_hbm, src_hbm, dst_hbm, acc_out, *rest):
        if with_cnt:
            cnt_out, acc_sh, idx_s, idx_d, rows, zbuf, sem, cnt_sh, ones_v = rest
        else:
            acc_sh, idx_s, idx_d, rows, zbuf, sem = rest
        cid = lax.axis_index("c")
        sid = lax.axis_index("s")
        wid = sid * 2 + cid

        z16 = jnp.zeros((16,), jnp.float32)
        for i in range(16):
            for k in range(D // 16):
                zbuf[i, pl.ds(k * 16, 16)] = z16
        if with_cnt:
            o16 = jnp.ones((16,), jnp.float32)
            for k in range(CHUNK // 16):
                ones_v[pl.ds(k * 16, 16)] = o16

        r0 = sid * ROWS_PER_TILE

        def zero_body(i, _):
            pltpu.sync_copy(zbuf, acc_sh.at[pl.ds(r0 + i * 16, 16)])
            return 0
        lax.fori_loop(0, ROWS_PER_TILE // 16, zero_body, 0)
        if with_cnt:
            def zero_cnt(i, _):
                pltpu.sync_copy(zbuf.at[0], cnt_sh.at[pl.ds(r0 + i * D, D)])
                return 0
            lax.fori_loop(0, ROWS_PER_TILE // D, zero_cnt, 0)

        plsc.subcore_barrier()

        ebase = wid * EDGES_PER_TILE

        def edge_body(j, _):
            off = ebase + j * CHUNK
            pltpu.sync_copy(src_hbm.at[pl.ds(off, CHUNK)], idx_s.at[0])
            pltpu.sync_copy(dst_hbm.at[pl.ds(off, CHUNK)], idx_d.at[0])
            pltpu.async_copy(h_hbm.at[idx_s.at[0]], rows, sem).wait()
            pltpu.sync_copy(rows, acc_sh.at[idx_d.at[0]], add=True)
            if with_cnt:
                pltpu.sync_copy(ones_v, cnt_sh.at[idx_d.at[0]], add=True)
            return 0
        lax.fori_loop(0, CHUNKS_PER_TILE, edge_body, 0)

        plsc.subcore_barrier()

        pltpu.sync_copy(acc_sh.at[pl.ds(r0, ROWS_PER_TILE)],
                        acc_out.at[cid].at[pl.ds(r0, ROWS_PER_TILE)])
        if with_cnt:
            pltpu.sync_copy(cnt_sh.at[pl.ds(r0, ROWS_PER_TILE)],
                            cnt_out.at[cid].at[pl.ds(r0, ROWS_PER_TILE)])

    return pl.kernel(body, out_type=out_type, mesh=mesh,
                     scratch_types=scratch)


_sc_agg_cnt = _make_sc_agg(True)
_sc_agg = _make_sc_agg(False)

_TC_ROWS = 1000


def _tc_layer_body(acc_ref, cnt_ref, h_ref, wl_ref, wr_ref, b_ref, out_ref):
    c = cnt_ref[0, :, 0] + cnt_ref[1, :, 0]
    s = acc_ref[0] + acc_ref[1]
    mean = s / jnp.maximum(c, 1.0)[:, None]
    o = jnp.dot(mean, wl_ref[...], preferred_element_type=jnp.float32)
    o = o + jnp.dot(h_ref[...], wr_ref[...], preferred_element_type=jnp.float32)
    o = o + b_ref[...]
    out_ref[...] = jnp.maximum(o, 0.0)


def _tc_layer(acc, cnt3, h, W_l, W_r, b):
    grid = (N // _TC_ROWS,)
    return pl.pallas_call(
        _tc_layer_body,
        grid=grid,
        in_specs=[
            pl.BlockSpec((2, _TC_ROWS, D), lambda i: (0, i, 0)),
            pl.BlockSpec((2, _TC_ROWS, 1), lambda i: (0, i, 0)),
            pl.BlockSpec((_TC_ROWS, D), lambda i: (i, 0)),
            pl.BlockSpec((D, D), lambda i: (0, 0)),
            pl.BlockSpec((D, D), lambda i: (0, 0)),
            pl.BlockSpec((1, D), lambda i: (0, 0)),
        ],
        out_specs=pl.BlockSpec((_TC_ROWS, D), lambda i: (i, 0)),
        out_shape=jax.ShapeDtypeStruct((N, D), jnp.float32),
    )(acc, cnt3, h, W_l, W_r, b.reshape(1, D))


def kernel(x, edge_index, W1_l, b1, W1_r, W2_l, b2, W2_r):
    pad = E_PAD - E
    src_p = jnp.concatenate([edge_index[0], jnp.zeros((pad,), jnp.int32)])
    dst_p = jnp.concatenate([edge_index[1],
                             jnp.full((pad,), DUMMY_ROW, jnp.int32)])

    acc1, cnt = _sc_agg_cnt(x, src_p, dst_p)
    cnt3 = cnt.reshape(2, N_PAD, 1)
    h = _tc_layer(acc1, cnt3, x, W1_l, W1_r, b1)
    acc2 = _sc_agg(h, src_p, dst_p)
    out = _tc_layer(acc2, cnt3, h, W2_l, W2_r, b2)
    return out


# speedup vs baseline: 4.4146x; 4.4146x over previous
import jax
import jax.numpy as jnp
from jax import lax
from jax.experimental import pallas as pl
from jax.experimental.pallas import tpu as pltpu
from jax.experimental.pallas import tpu_sc as plsc

N = 10000
E = 320000
D = 128

N_PAD = 10240
DUMMY_ROW = N
CHUNK = 128
N_WORKERS = 32
CHUNKS_PER_TILE = 79
EDGES_PER_TILE = CHUNKS_PER_TILE * CHUNK
E_PAD = N_WORKERS * EDGES_PER_TILE
ROWS_PER_TILE = N_PAD // 16


def _make_sc_agg(with_cnt: bool):
    mesh = plsc.VectorSubcoreMesh(core_axis_name="c", subcore_axis_name="s")
    out_type = [jax.ShapeDtypeStruct((2, N_PAD, D), jnp.float32)]
    scratch = [
        pltpu.VMEM_SHARED((N_PAD, D), jnp.float32),
        pltpu.VMEM((1, CHUNK), jnp.int32),
        pltpu.VMEM((1, CHUNK), jnp.int32),
        pltpu.VMEM((CHUNK, D), jnp.float32),
        pltpu.VMEM((16, D), jnp.float32),
        pltpu.SemaphoreType.DMA,
    ]
    if with_cnt:
        out_type.append(jax.ShapeDtypeStruct((2, N_PAD), jnp.float32))
        scratch += [
            pltpu.VMEM_SHARED((N_PAD,), jnp.float32),
            pltpu.VMEM((CHUNK,), jnp.float32),
        ]

    def body(h_hbm, src_hbm, dst_hbm, acc_out, *rest):
        if with_cnt:
            cnt_out, acc_sh, idx_s, idx_d, rows, zbuf, sem, cnt_sh, ones_v = rest
        else:
            acc_sh, idx_s, idx_d, rows, zbuf, sem = rest
        cid = lax.axis_index("c")
        sid = lax.axis_index("s")
        wid = sid * 2 + cid

        z16 = jnp.zeros((16,), jnp.float32)
        for i in range(16):
            for k in range(D // 16):
                zbuf[i, pl.ds(k * 16, 16)] = z16
        if with_cnt:
            o16 = jnp.ones((16,), jnp.float32)
            for k in range(CHUNK // 16):
                ones_v[pl.ds(k * 16, 16)] = o16

        r0 = sid * ROWS_PER_TILE

        def zero_body(i, _):
            pltpu.sync_copy(zbuf, acc_sh.at[pl.ds(r0 + i * 16, 16)])
            return 0
        lax.fori_loop(0, ROWS_PER_TILE // 16, zero_body, 0)
        if with_cnt:
            def zero_cnt(i, _):
                pltpu.sync_copy(zbuf.at[0], cnt_sh.at[pl.ds(r0 + i * D, D)])
                return 0
            lax.fori_loop(0, ROWS_PER_TILE // D, zero_cnt, 0)

        plsc.subcore_barrier()

        ebase = wid * EDGES_PER_TILE

        def edge_body(j, _):
            off = ebase + j * CHUNK
            pltpu.sync_copy(src_hbm.at[pl.ds(off, CHUNK)], idx_s.at[0])
            pltpu.sync_copy(dst_hbm.at[pl.ds(off, CHUNK)], idx_d.at[0])
            pltpu.async_copy(h_hbm.at[idx_s.at[0]], rows, sem).wait()
            pltpu.sync_copy(rows, acc_sh.at[idx_d.at[0]], add=True)
            if with_cnt:
                pltpu.sync_copy(ones_v, cnt_sh.at[idx_d.at[0]], add=True)
            return 0
        lax.fori_loop(0, CHUNKS_PER_TILE, edge_body, 0)

        plsc.subcore_barrier()

        pltpu.sync_copy(acc_sh.at[pl.ds(r0, ROWS_PER_TILE)],
                        acc_out.at[cid].at[pl.ds(r0, ROWS_PER_TILE)])
        if with_cnt:
            pltpu.sync_copy(cnt_sh.at[pl.ds(r0, ROWS_PER_TILE)],
                            cnt_out.at[cid].at[pl.ds(r0, ROWS_PER_TILE)])

    return pl.kernel(body, out_type=out_type, mesh=mesh,
                     scratch_types=scratch)


_sc_agg_cnt = _make_sc_agg(True)
_sc_agg = _make_sc_agg(False)

_TC_ROWS = 1000


def _tc_layer_body(acc_ref, cnt_ref, h_ref, wl_ref, wr_ref, b_ref, out_ref):
    c = cnt_ref[0, :, 0] + cnt_ref[1, :, 0]
    s = acc_ref[0] + acc_ref[1]
    mean = s / jnp.maximum(c, 1.0)[:, None]
    o = jnp.dot(mean, wl_ref[...], preferred_element_type=jnp.float32)
    o = o + jnp.dot(h_ref[...], wr_ref[...], preferred_element_type=jnp.float32)
    o = o + b_ref[...]
    out_ref[...] = jnp.maximum(o, 0.0)


def _tc_layer(acc, cnt3, h, W_l, W_r, b):
    grid = (N // _TC_ROWS,)
    return pl.pallas_call(
        _tc_layer_body,
        grid=grid,
        in_specs=[
            pl.BlockSpec((2, _TC_ROWS, D), lambda i: (0, i, 0)),
            pl.BlockSpec((2, _TC_ROWS, 1), lambda i: (0, i, 0)),
            pl.BlockSpec((_TC_ROWS, D), lambda i: (i, 0)),
            pl.BlockSpec((D, D), lambda i: (0, 0)),
            pl.BlockSpec((D, D), lambda i: (0, 0)),
            pl.BlockSpec((1, D), lambda i: (0, 0)),
        ],
        out_specs=pl.BlockSpec((_TC_ROWS, D), lambda i: (i, 0)),
        out_shape=jax.ShapeDtypeStruct((N, D), jnp.float32),
    )(acc, cnt3, h, W_l, W_r, b.reshape(1, D))


def kernel(x, edge_index, W1_l, b1, W1_r, W2_l, b2, W2_r):
    pad = E_PAD - E
    src_p = jnp.concatenate([edge_index[0], jnp.zeros((pad,), jnp.int32)])
    dst_p = jnp.concatenate([edge_index[1],
                             jnp.full((pad,), DUMMY_ROW, jnp.int32)])

    acc1, cnt = _sc_agg_cnt(x, src_p, dst_p)
    cnt3 = cnt.reshape(2, N_PAD, 1)
    h = _tc_layer(acc1, cnt3, x, W1_l, W1_r, b1)
    (acc2,) = _sc_agg(h, src_p, dst_p)
    out = _tc_layer(acc2, cnt3, h, W2_l, W2_r, b2)
    return out
